# Initial kernel scaffold; baseline (speedup 1.0000x reference)
import jax
import jax.numpy as jnp
from jax.experimental import pallas as pl


def kernel(up_w, up_b, dw_w, dw_b, pw_w, pw_b, bn_gamma, bn_beta, bn_mean, bn_var, x, skip):
    raise NotImplementedError("write your pallas kernel here")


# speedup vs baseline: 1.2754x; 1.2754x over previous
import numpy as np
import jax
import jax.numpy as jnp
from jax import lax
from jax.experimental import pallas as pl
from jax.experimental.pallas import tpu as pltpu

_VMEM_LIMIT = 64 * 1024 * 1024


def _resize_matrix(out_size, in_size):
    scale = in_size / out_size
    i = np.arange(out_size, dtype=np.float64)
    src = np.maximum((i + 0.5) * scale - 0.5, 0.0)
    i0 = np.minimum(np.floor(src).astype(np.int64), in_size - 1)
    i1 = np.minimum(i0 + 1, in_size - 1)
    lam1 = (src - i0).astype(np.float32)
    R = np.zeros((out_size, in_size), np.float32)
    R[np.arange(out_size), i0] += 1.0 - lam1
    R[np.arange(out_size), i1] += lam1
    return R


def _fused_kernel(x_ref, sk_ref, wup_ref, bup_ref, rh_ref, rw_ref,
                  wdw_ref, bdw_ref, wpw_ref, bpw_ref, o_ref, xs_ref):
    Cin = x_ref.shape[1]
    Cs = sk_ref.shape[1]
    Hp, Wp, C2 = xs_ref.shape
    Ho, Wo = Hp - 2, Wp - 2
    H, W = Ho // 2, Wo // 2
    Cout = o_ref.shape[1]
    Hs = rh_ref.shape[1]
    Ws = rw_ref.shape[1]

    xm = x_ref[0]
    up_flat = lax.dot_general(
        xm, wup_ref[...], (((0,), (0,)), ((), ())),
        preferred_element_type=jnp.float32)

    u00 = up_flat[:, 0 * Cout:1 * Cout].reshape(H, W, 1, Cout)
    u01 = up_flat[:, 1 * Cout:2 * Cout].reshape(H, W, 1, Cout)
    u10 = up_flat[:, 2 * Cout:3 * Cout].reshape(H, W, 1, Cout)
    u11 = up_flat[:, 3 * Cout:4 * Cout].reshape(H, W, 1, Cout)
    r0 = jnp.concatenate([u00, u01], axis=2).reshape(H, 1, Wo, Cout)
    r1 = jnp.concatenate([u10, u11], axis=2).reshape(H, 1, Wo, Cout)
    up_img = jnp.concatenate([r0, r1], axis=1).reshape(Ho, Wo, Cout)
    up_img = up_img + bup_ref[...].reshape(1, 1, Cout)

    sT = sk_ref[0].T
    s3 = sT.reshape(Hs, Ws * Cs)
    th = jnp.dot(rh_ref[...], s3,
                 preferred_element_type=jnp.float32)
    th = th.reshape(Ho, Ws, Cs)
    th = jnp.transpose(th, (0, 2, 1))
    tw = jnp.dot(th.reshape(Ho * Cs, Ws), rw_ref[...].T,
                 preferred_element_type=jnp.float32)
    sk_img = jnp.transpose(tw.reshape(Ho, Cs, Wo), (0, 2, 1))

    xs_ref[0:1, :, :] = jnp.zeros((1, Wp, C2), jnp.float32)
    xs_ref[Hp - 1:Hp, :, :] = jnp.zeros((1, Wp, C2), jnp.float32)
    xs_ref[:, 0:1, :] = jnp.zeros((Hp, 1, C2), jnp.float32)
    xs_ref[:, Wp - 1:Wp, :] = jnp.zeros((Hp, 1, C2), jnp.float32)
    xs_ref[1:Ho + 1, 1:Wo + 1, 0:Cout] = up_img
    xs_ref[1:Ho + 1, 1:Wo + 1, Cout:C2] = sk_img

    wdw = wdw_ref[...]
    acc = jnp.zeros((Ho, Wo, C2), jnp.float32)
    for kh in range(3):
        for kw in range(3):
            acc = acc + xs_ref[kh:kh + Ho, kw:kw + Wo, :] * wdw[kh, kw, :]
    acc = acc + bdw_ref[...].reshape(1, 1, C2)

    y = jnp.dot(acc.reshape(Ho * Wo, C2), wpw_ref[...],
                preferred_element_type=jnp.float32) + bpw_ref[...]
    y = jnp.maximum(y, 0.0)
    o_ref[0] = y.T


def kernel(up_w, up_b, dw_w, dw_b, pw_w, pw_b,
           bn_gamma, bn_beta, bn_mean, bn_var, x, skip):
    N, Cin, H, W = x.shape
    _, Cs, Hs, Ws = skip.shape
    Cout = up_w.shape[1]
    C2 = 2 * Cout
    Ho, Wo = 2 * H, 2 * W

    wup = jnp.transpose(up_w, (0, 2, 3, 1)).reshape(Cin, 4 * Cout)
    bup = up_b.reshape(1, Cout)
    rh = jnp.asarray(_resize_matrix(Ho, Hs))
    rw = jnp.asarray(_resize_matrix(Wo, Ws))
    wdw = jnp.transpose(dw_w[:, 0, :, :], (1, 2, 0))
    bdw = dw_b.reshape(1, C2)
    inv = bn_gamma / jnp.sqrt(bn_var + 1e-5)
    wpw = jnp.transpose(pw_w[:, :, 0, 0], (1, 0)) * inv[None, :]
    bpw = (pw_b * inv + bn_beta - bn_mean * inv).reshape(1, Cout)

    x_flat = x.reshape(N, Cin, H * W)
    sk_flat = skip.reshape(N, Cs, Hs * Ws)

    out = pl.pallas_call(
        _fused_kernel,
        out_shape=jax.ShapeDtypeStruct((N, Cout, Ho * Wo), jnp.float32),
        grid=(N,),
        in_specs=[
            pl.BlockSpec((1, Cin, H * W), lambda n: (n, 0, 0)),
            pl.BlockSpec((1, Cs, Hs * Ws), lambda n: (n, 0, 0)),
            pl.BlockSpec((Cin, 4 * Cout), lambda n: (0, 0)),
            pl.BlockSpec((1, Cout), lambda n: (0, 0)),
            pl.BlockSpec((Ho, Hs), lambda n: (0, 0)),
            pl.BlockSpec((Wo, Ws), lambda n: (0, 0)),
            pl.BlockSpec((3, 3, C2), lambda n: (0, 0, 0)),
            pl.BlockSpec((1, C2), lambda n: (0, 0)),
            pl.BlockSpec((C2, Cout), lambda n: (0, 0)),
            pl.BlockSpec((1, Cout), lambda n: (0, 0)),
        ],
        out_specs=pl.BlockSpec((1, Cout, Ho * Wo), lambda n: (n, 0, 0)),
        scratch_shapes=[pltpu.VMEM((Ho + 2, Wo + 2, C2), jnp.float32)],
        compiler_params=pltpu.CompilerParams(
            dimension_semantics=("parallel",),
            vmem_limit_bytes=_VMEM_LIMIT),
    )(x_flat, sk_flat, wup, bup, rh, rw, wdw, bdw, wpw, bpw)

    return out.reshape(N, Cout, Ho, Wo)
